# baseline (speedup 1.0000x reference)
import functools

import jax
import jax.numpy as jnp
from jax import lax
from jax.experimental import pallas as pl
from jax.experimental.pallas import tpu as pltpu
from jax.experimental.pallas import tpu_sc as plsc

_T = 4096
_D = 1024
_NC = 2
_RPC = _T // _NC
_SCH = 256
_SNB = 4


def _make_scs_copy():
    mesh = plsc.ScalarSubcoreMesh(axis_name="c", num_cores=_NC)
    n = _RPC // _SCH

    @functools.partial(
        pl.kernel,
        mesh=mesh,
        out_type=jax.ShapeDtypeStruct((_T, _D), jnp.float32),
        scratch_types=[
            pltpu.VMEM_SHARED((_SNB, _SCH, _D), jnp.float32),
            *([pltpu.SemaphoreType.DMA] * (2 * _SNB)),
        ],
    )
    def scs_copy(table_hbm, out_hbm, stage, *sems):
        in_sems = sems[:_SNB]
        out_sems = sems[_SNB:]
        base = lax.axis_index("c") * _RPC

        def fire_in(k):
            b = k % _SNB
            return pltpu.async_copy(
                table_hbm.at[pl.ds(base + k * _SCH, _SCH)], stage.at[b], in_sems[b]
            )

        def fire_out(k):
            b = k % _SNB
            return pltpu.async_copy(
                stage.at[b], out_hbm.at[pl.ds(base + k * _SCH, _SCH)], out_sems[b]
            )

        in_cp = [None] * n
        out_cp = [None] * n
        for j in range(min(_SNB, n)):
            in_cp[j] = fire_in(j)
        for k in range(n):
            if k >= _SNB:
                out_cp[k - _SNB].wait()
                in_cp[k] = fire_in(k)
            in_cp[k].wait()
            out_cp[k] = fire_out(k)
        for k in range(max(0, n - _SNB), n):
            out_cp[k].wait()

    return scs_copy


_scs_copy = _make_scs_copy()


def kernel(x, pe_table):
    del x
    out = _scs_copy(pe_table)
    return out[None]
